# baseline (speedup 1.0000x reference)
import functools

import jax
import jax.numpy as jnp
from jax import lax
from jax.experimental import pallas as pl
from jax.experimental.pallas import tpu as pltpu
from jax.experimental.pallas import tpu_sc as plsc

B = 256
S = 512
H = 768
L = 16
NC, NS = 2, 16
NW = NC * NS
PB = S // NW
SEQ_C = 4
T_C = SEQ_C * PB
N_CHUNK = B // SEQ_C
HJ = H // L
EPS = 1e-12

_mesh = plsc.VectorSubcoreMesh(core_axis_name="c", subcore_axis_name="s")


def _lane_sum(x):
    ii = lax.iota(jnp.int32, L)
    for sh in (8, 4, 2, 1):
        idx = lax.bitwise_xor(ii, jnp.int32(sh))
        dnums = lax.GatherDimensionNumbers(
            offset_dims=(), collapsed_slice_dims=(0,), start_index_map=(0,))
        x = x + lax.gather(
            x, idx[:, None], dnums, slice_sizes=(1,),
            mode=lax.GatherScatterMode.PROMISE_IN_BOUNDS)
    return x


def _rsqrt(v):
    i = lax.bitcast_convert_type(v, jnp.int32)
    i = jnp.int32(0x5F3759DF) - lax.shift_right_arithmetic(i, jnp.int32(1))
    y = lax.bitcast_convert_type(i, jnp.float32)
    for _ in range(3):
        y = y * (1.5 - 0.5 * v * y * y)
    return y


@functools.partial(
    pl.kernel,
    out_type=jax.ShapeDtypeStruct((B, S, H), jnp.float32),
    mesh=_mesh,
    scratch_types=[
        pltpu.VMEM((T_C,), jnp.int32),
        pltpu.VMEM((T_C, H), jnp.float32),
        pltpu.VMEM((PB, H), jnp.float32),
        pltpu.VMEM((H,), jnp.float32),
        pltpu.VMEM((H,), jnp.float32),
        pltpu.VMEM((H,), jnp.float32),
        pltpu.SemaphoreType.DMA,
    ],
)
def _bert_emb(ids_hbm, word_hbm, pos_hbm, type_hbm, gamma_hbm, beta_hbm,
              out_hbm, idx_v, rows_v, pos_v, type_v, gamma_v, beta_v, sem):
    w = lax.axis_index("s") * NC + lax.axis_index("c")

    pltpu.sync_copy(pos_hbm.at[pl.ds(w * PB, PB), :], pos_v)
    pltpu.sync_copy(type_hbm.at[0], type_v)
    pltpu.sync_copy(gamma_hbm, gamma_v)
    pltpu.sync_copy(beta_hbm, beta_v)

    def _add_type(i, _):
        for j in range(HJ):
            sl = pl.ds(j * L, L)
            pos_v[i, sl] = pos_v[i, sl] + type_v[sl]
        return 0

    lax.fori_loop(0, PB, _add_type, 0)

    def _row_ln(t, _):
        p = lax.bitwise_and(t, PB - 1)
        s_acc = jnp.zeros((L,), jnp.float32)
        q_acc = jnp.zeros((L,), jnp.float32)
        for j in range(HJ):
            sl = pl.ds(j * L, L)
            x = rows_v[t, sl] + pos_v[p, sl]
            rows_v[t, sl] = x
            s_acc = s_acc + x
            q_acc = q_acc + x * x
        mean_v = _lane_sum(s_acc) * (1.0 / H)
        var_v = _lane_sum(q_acc) * (1.0 / H) - mean_v * mean_v
        rstd_v = _rsqrt(var_v + EPS)
        for j in range(HJ):
            sl = pl.ds(j * L, L)
            x = rows_v[t, sl]
            rows_v[t, sl] = (x - mean_v) * rstd_v * gamma_v[sl] + beta_v[sl]
        return 0

    def _chunk(c, _):
        pltpu.sync_copy(ids_hbm.at[w, pl.ds(c * T_C, T_C)], idx_v)
        pltpu.async_copy(word_hbm.at[idx_v], rows_v, sem).wait()
        lax.fori_loop(0, T_C, _row_ln, 0)
        for s_l in range(SEQ_C):
            pltpu.sync_copy(
                rows_v.at[pl.ds(s_l * PB, PB), :],
                out_hbm.at[c * SEQ_C + s_l, pl.ds(w * PB, PB), :],
            )
        return 0

    lax.fori_loop(0, N_CHUNK, _chunk, 0)


def kernel(input_ids, word_emb, pos_emb, type_emb, gamma, beta):
    ids_r = (
        input_ids.astype(jnp.int32)
        .reshape(B, NW, PB)
        .transpose(1, 0, 2)
        .reshape(NW, B * PB)
    )
    return _bert_emb(ids_r, word_emb, pos_emb, type_emb, gamma, beta)


# speedup vs baseline: 1.1390x; 1.1390x over previous
import functools

import jax
import jax.numpy as jnp
from jax import lax
from jax.experimental import pallas as pl
from jax.experimental.pallas import tpu as pltpu
from jax.experimental.pallas import tpu_sc as plsc

B = 256
S = 512
H = 768
L = 16
NC, NS = 2, 16
NW = NC * NS
PB = S // NW
SEQ_C = 2
T_C = SEQ_C * PB
N_CHUNK = B // SEQ_C
NBUF = 3
HJ = H // L
EPS = 1e-12

_mesh = plsc.VectorSubcoreMesh(core_axis_name="c", subcore_axis_name="s")


def _lane_sum(x):
    ii = lax.iota(jnp.int32, L)
    dnums = lax.GatherDimensionNumbers(
        offset_dims=(), collapsed_slice_dims=(0,), start_index_map=(0,))
    for sh in (8, 4, 2, 1):
        idx = lax.bitwise_xor(ii, jnp.int32(sh))
        x = x + lax.gather(
            x, idx[:, None], dnums, slice_sizes=(1,),
            mode=lax.GatherScatterMode.PROMISE_IN_BOUNDS)
    return x


def _rsqrt(v):
    i = lax.bitcast_convert_type(v, jnp.int32)
    i = jnp.int32(0x5F3759DF) - lax.shift_right_arithmetic(i, jnp.int32(1))
    y = lax.bitcast_convert_type(i, jnp.float32)
    for _ in range(3):
        y = y * (1.5 - 0.5 * v * y * y)
    return y


@functools.partial(
    pl.kernel,
    out_type=jax.ShapeDtypeStruct((B, S, H), jnp.float32),
    mesh=_mesh,
    scratch_types=[
        pltpu.VMEM((B * PB,), jnp.int32),
        [pltpu.VMEM((T_C, H), jnp.float32) for _ in range(NBUF)],
        pltpu.VMEM((PB, H), jnp.float32),
        pltpu.VMEM((H,), jnp.float32),
        pltpu.VMEM((H,), jnp.float32),
        pltpu.VMEM((H,), jnp.float32),
        [pltpu.SemaphoreType.DMA for _ in range(NBUF)],
        [pltpu.SemaphoreType.DMA for _ in range(NBUF)],
    ],
)
def _bert_emb(ids_hbm, word_hbm, pos_hbm, type_hbm, gamma_hbm, beta_hbm,
              out_hbm, ids_v, bufs, pos_v, type_v, gamma_v, beta_v,
              gsem, osem):
    w = lax.axis_index("s") * NC + lax.axis_index("c")

    pltpu.sync_copy(pos_hbm.at[pl.ds(w * PB, PB), :], pos_v)
    pltpu.sync_copy(type_hbm.at[0], type_v)
    pltpu.sync_copy(gamma_hbm, gamma_v)
    pltpu.sync_copy(beta_hbm, beta_v)
    pltpu.sync_copy(ids_hbm.at[w], ids_v)

    def _add_type(i, _):
        for j in range(HJ):
            sl = pl.ds(j * L, L)
            pos_v[i, sl] = pos_v[i, sl] + type_v[sl]
        return 0

    lax.fori_loop(0, PB, _add_type, 0)

    def _gather_start(c, b):
        idx = ids_v.at[pl.ds(c * T_C, T_C)]
        pltpu.async_copy(word_hbm.at[idx], bufs[b], gsem[b])

    def _gather_wait(b):
        pltpu.make_async_copy(word_hbm.at[ids_v.at[pl.ds(0, T_C)]],
                              bufs[b], gsem[b]).wait()

    def _out_wait(b):
        for _ in range(SEQ_C):
            pltpu.make_async_copy(
                bufs[b].at[pl.ds(0, PB), :],
                out_hbm.at[0, pl.ds(w * PB, PB), :],
                osem[b]).wait()

    def _row_ln(rows_v, t, _):
        p = lax.bitwise_and(t, PB - 1)
        s_acc = jnp.zeros((L,), jnp.float32)
        q_acc = jnp.zeros((L,), jnp.float32)
        for j in range(HJ):
            sl = pl.ds(j * L, L)
            x = rows_v[t, sl] + pos_v[p, sl]
            rows_v[t, sl] = x
            s_acc = s_acc + x
            q_acc = q_acc + x * x
        mean_v = _lane_sum(s_acc) * (1.0 / H)
        var_v = _lane_sum(q_acc) * (1.0 / H) - mean_v * mean_v
        rstd_v = _rsqrt(var_v + EPS)
        for j in range(HJ):
            sl = pl.ds(j * L, L)
            x = rows_v[t, sl]
            rows_v[t, sl] = (x - mean_v) * rstd_v * gamma_v[sl] + beta_v[sl]
        return 0

    _gather_start(0, 0)

    def _outer(it, _):
        c0 = it * NBUF
        for b in range(NBUF):
            c = c0 + b
            bn = (b + 1) % NBUF

            @pl.when(c + 1 < N_CHUNK)
            def _():
                @pl.when(c >= 2)
                def _():
                    _out_wait(bn)
                _gather_start(c + 1, bn)

            @pl.when(c < N_CHUNK)
            def _():
                _gather_wait(b)
                lax.fori_loop(0, T_C, functools.partial(_row_ln, bufs[b]), 0)
                for s_l in range(SEQ_C):
                    pltpu.async_copy(
                        bufs[b].at[pl.ds(s_l * PB, PB), :],
                        out_hbm.at[c * SEQ_C + s_l, pl.ds(w * PB, PB), :],
                        osem[b])
        return 0

    n_outer = (N_CHUNK + NBUF - 1) // NBUF
    lax.fori_loop(0, n_outer, _outer, 0)

    for b in range(NBUF):
        _out_wait(b)


def kernel(input_ids, word_emb, pos_emb, type_emb, gamma, beta):
    ids_r = (
        input_ids.astype(jnp.int32)
        .reshape(B, NW, PB)
        .transpose(1, 0, 2)
        .reshape(NW, B * PB)
    )
    return _bert_emb(ids_r, word_emb, pos_emb, type_emb, gamma, beta)


# speedup vs baseline: 3.6336x; 3.1901x over previous
import functools

import jax
import jax.numpy as jnp
from jax import lax
from jax.experimental import pallas as pl
from jax.experimental.pallas import tpu as pltpu
from jax.experimental.pallas import tpu_sc as plsc

B = 256
S = 512
H = 768
L = 16
NC, NS = 2, 16
NW = NC * NS
PB = S // NW
SEQ_C = 2
T_C = SEQ_C * PB
N_CHUNK = B // SEQ_C
NBUF = 2
HJ = H // L
EPS = 1e-12

_mesh = plsc.VectorSubcoreMesh(core_axis_name="c", subcore_axis_name="s")


def _lane_sum2(a, b):
    ii = lax.iota(jnp.int32, L)
    dnums = lax.GatherDimensionNumbers(
        offset_dims=(), collapsed_slice_dims=(0,), start_index_map=(0,))

    def perm(x, idx):
        return lax.gather(x, idx[:, None], dnums, slice_sizes=(1,),
                          mode=lax.GatherScatterMode.PROMISE_IN_BOUNDS)

    for sh in (8, 4, 2, 1):
        idx = lax.bitwise_xor(ii, jnp.int32(sh))
        a = a + perm(a, idx)
        b = b + perm(b, idx)
    return a, b


def _rsqrt(v):
    i = lax.bitcast_convert_type(v, jnp.int32)
    i = jnp.int32(0x5F3759DF) - lax.shift_right_arithmetic(i, jnp.int32(1))
    y = lax.bitcast_convert_type(i, jnp.float32)
    for _ in range(2):
        y = y * (1.5 - 0.5 * v * y * y)
    return y


@functools.partial(
    pl.kernel,
    out_type=jax.ShapeDtypeStruct((B, S, H), jnp.float32),
    mesh=_mesh,
    scratch_types=[
        pltpu.VMEM((B * PB,), jnp.int32),
        [pltpu.VMEM((T_C, H), jnp.float32) for _ in range(NBUF)],
        [pltpu.VMEM((T_C, H), jnp.float32) for _ in range(NBUF)],
        pltpu.VMEM((2 * T_C, L), jnp.float32),
        pltpu.VMEM((PB, H), jnp.float32),
        pltpu.VMEM((H,), jnp.float32),
        pltpu.VMEM((H,), jnp.float32),
        pltpu.VMEM((H,), jnp.float32),
        [pltpu.SemaphoreType.DMA for _ in range(NBUF)],
        [pltpu.SemaphoreType.DMA for _ in range(NBUF)],
    ],
)
def _bert_emb(ids_hbm, word_hbm, pos_hbm, type_hbm, gamma_hbm, beta_hbm,
              out_hbm, ids_v, gbufs, ybufs, stats_v, pos_v, type_v,
              gamma_v, beta_v, gsem, osem):
    w = lax.axis_index("s") * NC + lax.axis_index("c")

    pltpu.sync_copy(pos_hbm.at[pl.ds(w * PB, PB), :], pos_v)
    pltpu.sync_copy(type_hbm.at[0], type_v)
    pltpu.sync_copy(gamma_hbm, gamma_v)
    pltpu.sync_copy(beta_hbm, beta_v)
    pltpu.sync_copy(ids_hbm.at[w], ids_v)

    def _add_type(i, _):
        for j in range(HJ):
            sl = pl.ds(j * L, L)
            pos_v[i, sl] = pos_v[i, sl] + type_v[sl]
        return 0

    lax.fori_loop(0, PB, _add_type, 0)

    def _gather_start(c, b):
        idx = ids_v.at[pl.ds(c * T_C, T_C)]
        pltpu.async_copy(word_hbm.at[idx], gbufs[b], gsem[b])

    def _gather_wait(b):
        pltpu.make_async_copy(word_hbm.at[ids_v.at[pl.ds(0, T_C)]],
                              gbufs[b], gsem[b]).wait()

    def _out_wait(b):
        for _ in range(SEQ_C):
            pltpu.make_async_copy(
                ybufs[b].at[pl.ds(0, PB), :],
                out_hbm.at[0, pl.ds(w * PB, PB), :],
                osem[b]).wait()

    def _stats_pair(rows_v, t, _):
        u = t + PB
        s0 = jnp.zeros((L,), jnp.float32)
        q0 = jnp.zeros((L,), jnp.float32)
        s1 = jnp.zeros((L,), jnp.float32)
        q1 = jnp.zeros((L,), jnp.float32)
        for j in range(HJ):
            sl = pl.ds(j * L, L)
            p = pos_v[t, sl]
            x0 = rows_v[t, sl] + p
            x1 = rows_v[u, sl] + p
            s0 = s0 + x0
            q0 = q0 + x0 * x0
            s1 = s1 + x1
            q1 = q1 + x1 * x1
        s0, s1 = _lane_sum2(s0, s1)
        q0, q1 = _lane_sum2(q0, q1)
        m0 = s0 * (1.0 / H)
        m1 = s1 * (1.0 / H)
        a0 = _rsqrt(q0 * (1.0 / H) - m0 * m0 + EPS)
        a1 = _rsqrt(q1 * (1.0 / H) - m1 * m1 + EPS)
        stats_v[2 * t, :] = m0
        stats_v[2 * t + 1, :] = a0
        stats_v[2 * u, :] = m1
        stats_v[2 * u + 1, :] = a1
        return 0

    QB = 12

    def _norm_pair(rows_v, ybuf, t, _):
        u = t + PB
        m0 = stats_v[2 * t, :]
        a0 = stats_v[2 * t + 1, :]
        m1 = stats_v[2 * u, :]
        a1 = stats_v[2 * u + 1, :]
        for j0 in range(0, HJ, QB):
            ys = []
            for j in range(j0, j0 + QB):
                sl = pl.ds(j * L, L)
                p = pos_v[t, sl]
                g = gamma_v[sl]
                bt = beta_v[sl]
                pm0 = p - m0
                pm1 = p - m1
                ys.append((rows_v[t, sl] + pm0) * a0 * g + bt)
                ys.append((rows_v[u, sl] + pm1) * a1 * g + bt)
            for k, j in enumerate(range(j0, j0 + QB)):
                sl = pl.ds(j * L, L)
                ybuf[t, sl] = ys[2 * k]
                ybuf[u, sl] = ys[2 * k + 1]
        return 0

    _gather_start(0, 0)

    def _outer(it, _):
        c0 = it * NBUF
        for b in range(NBUF):
            c = c0 + b
            bn = (b + 1) % NBUF

            @pl.when(c + 1 < N_CHUNK)
            def _():
                _gather_start(c + 1, bn)

            @pl.when(c < N_CHUNK)
            def _():
                _gather_wait(b)

                @pl.when(c >= NBUF)
                def _():
                    _out_wait(b)
                lax.fori_loop(0, PB,
                              functools.partial(_stats_pair, gbufs[b]), 0)
                lax.fori_loop(0, PB,
                              functools.partial(_norm_pair, gbufs[b],
                                                ybufs[b]), 0)
                for s_l in range(SEQ_C):
                    pltpu.async_copy(
                        ybufs[b].at[pl.ds(s_l * PB, PB), :],
                        out_hbm.at[c * SEQ_C + s_l, pl.ds(w * PB, PB), :],
                        osem[b])
        return 0

    n_outer = (N_CHUNK + NBUF - 1) // NBUF
    lax.fori_loop(0, n_outer, _outer, 0)

    for b in range(NBUF):
        _out_wait(b)


def kernel(input_ids, word_emb, pos_emb, type_emb, gamma, beta):
    ids_r = (
        input_ids.astype(jnp.int32)
        .reshape(B, NW, PB)
        .transpose(1, 0, 2)
        .reshape(NW, B * PB)
    )
    return _bert_emb(ids_r, word_emb, pos_emb, type_emb, gamma, beta)


# speedup vs baseline: 3.9934x; 1.0990x over previous
import functools

import jax
import jax.numpy as jnp
from jax import lax
from jax.experimental import pallas as pl
from jax.experimental.pallas import tpu as pltpu
from jax.experimental.pallas import tpu_sc as plsc

B = 256
S = 512
H = 768
L = 16
NC, NS = 2, 16
NW = NC * NS
PB = S // NW
SEQ_C = 4
T_C = SEQ_C * PB
N_CHUNK = B // SEQ_C
NBUF = 2
HJ = H // L
TQ = 4
EPS = 1e-12

_mesh = plsc.VectorSubcoreMesh(core_axis_name="c", subcore_axis_name="s")


def _lane_sum2(a, b):
    ii = lax.iota(jnp.int32, L)
    dnums = lax.GatherDimensionNumbers(
        offset_dims=(), collapsed_slice_dims=(0,), start_index_map=(0,))

    def perm(x, idx):
        return lax.gather(x, idx[:, None], dnums, slice_sizes=(1,),
                          mode=lax.GatherScatterMode.PROMISE_IN_BOUNDS)

    for sh in (8, 4, 2, 1):
        idx = lax.bitwise_xor(ii, jnp.int32(sh))
        a = a + perm(a, idx)
        b = b + perm(b, idx)
    return a, b


def _rsqrt(v):
    i = lax.bitcast_convert_type(v, jnp.int32)
    i = jnp.int32(0x5F3759DF) - lax.shift_right_arithmetic(i, jnp.int32(1))
    y = lax.bitcast_convert_type(i, jnp.float32)
    for _ in range(2):
        y = y * (1.5 - 0.5 * v * y * y)
    return y


@functools.partial(
    pl.kernel,
    out_type=jax.ShapeDtypeStruct((B, S, H), jnp.float32),
    mesh=_mesh,
    scratch_types=[
        pltpu.VMEM((B * PB // 4,), jnp.int32),
        [pltpu.VMEM((T_C, H), jnp.float32) for _ in range(NBUF)],
        pltpu.VMEM((2 * T_C, L), jnp.float32),
        pltpu.VMEM((PB, H), jnp.float32),
        pltpu.VMEM((H,), jnp.float32),
        pltpu.VMEM((H,), jnp.float32),
        pltpu.VMEM((H,), jnp.float32),
        [pltpu.SemaphoreType.DMA for _ in range(NBUF)],
        [pltpu.SemaphoreType.DMA for _ in range(NBUF)],
    ],
)
def _bert_emb(ids_hbm, word_hbm, pos_hbm, type_hbm, gamma_hbm, beta_hbm,
              out_hbm, ids_v, gbufs, stats_v, pos_v, type_v,
              gamma_v, beta_v, gsem, osem):
    w = lax.axis_index("s") * NC + lax.axis_index("c")

    pltpu.sync_copy(pos_hbm.at[pl.ds(w * PB, PB), :], pos_v)
    pltpu.sync_copy(type_hbm.at[0], type_v)
    pltpu.sync_copy(gamma_hbm, gamma_v)
    pltpu.sync_copy(beta_hbm, beta_v)
    pltpu.sync_copy(ids_hbm.at[w * 4], ids_v)

    def _add_type(i, _):
        for j in range(HJ):
            sl = pl.ds(j * L, L)
            pos_v[i, sl] = pos_v[i, sl] + type_v[sl]
        return 0

    lax.fori_loop(0, PB, _add_type, 0)

    def _gather_start(c, b):
        off = lax.rem(c, N_CHUNK // 4) * T_C
        idx = ids_v.at[pl.ds(off, T_C)]
        pltpu.async_copy(word_hbm.at[idx], gbufs[b], gsem[b])

    def _gather_wait(b):
        pltpu.make_async_copy(word_hbm.at[ids_v.at[pl.ds(0, T_C)]],
                              gbufs[b], gsem[b]).wait()

    def _out_wait(b):
        for _ in range(SEQ_C * (PB // TQ)):
            pltpu.make_async_copy(
                gbufs[b].at[pl.ds(0, TQ), :],
                out_hbm.at[0, pl.ds(w * PB, TQ), :],
                osem[b]).wait()

    def _stats_quad(rows_v, t, _):
        rows = [t, t + PB, t + 2 * PB, t + 3 * PB]
        ss = [jnp.zeros((L,), jnp.float32) for _ in range(SEQ_C)]
        qs = [jnp.zeros((L,), jnp.float32) for _ in range(SEQ_C)]
        for j in range(HJ):
            sl = pl.ds(j * L, L)
            p = pos_v[t, sl]
            for r in range(SEQ_C):
                x = rows_v[rows[r], sl] + p
                ss[r] = ss[r] + x
                qs[r] = qs[r] + x * x
        ss[0], ss[1] = _lane_sum2(ss[0], ss[1])
        ss[2], ss[3] = _lane_sum2(ss[2], ss[3])
        qs[0], qs[1] = _lane_sum2(qs[0], qs[1])
        qs[2], qs[3] = _lane_sum2(qs[2], qs[3])
        for r in range(SEQ_C):
            m = ss[r] * (1.0 / H)
            a = _rsqrt(qs[r] * (1.0 / H) - m * m + EPS)
            stats_v[2 * rows[r], :] = m
            stats_v[2 * rows[r] + 1, :] = a
        return 0

    QB = 4

    def _norm_quad(rows_v, t, _):
        rows = [t, t + PB, t + 2 * PB, t + 3 * PB]
        ms = [stats_v[2 * r, :] for r in rows]
        as_ = [stats_v[2 * r + 1, :] for r in rows]
        for j0 in range(0, HJ, QB):
            ys = []
            for j in range(j0, j0 + QB):
                sl = pl.ds(j * L, L)
                p = pos_v[t, sl]
                g = gamma_v[sl]
                bt = beta_v[sl]
                for r in range(SEQ_C):
                    ys.append((rows_v[rows[r], sl] + (p - ms[r]))
                              * as_[r] * g + bt)
            for k, j in enumerate(range(j0, j0 + QB)):
                sl = pl.ds(j * L, L)
                for r in range(SEQ_C):
                    rows_v[rows[r], sl] = ys[SEQ_C * k + r]
        return 0

    _gather_start(0, 0)

    def _outer(it, _):
        c0 = it * NBUF
        for b in range(NBUF):
            c = c0 + b
            bn = (b + 1) % NBUF

            @pl.when(c + 1 < N_CHUNK)
            def _():
                @pl.when(c >= 1)
                def _():
                    _out_wait(bn)
                @pl.when(lax.rem(c + 1, N_CHUNK // 4) != 0)
                def _():
                    _gather_start(c + 1, bn)

            @pl.when(c < N_CHUNK)
            def _():
                _gather_wait(b)
                @pl.when(jnp.logical_and(lax.rem(c + 1, N_CHUNK // 4) == 0,
                                         c + 1 < N_CHUNK))
                def _():
                    q = lax.div(c + 1, N_CHUNK // 4)
                    pltpu.sync_copy(ids_hbm.at[w * 4 + q], ids_v)
                    _gather_start(c + 1, bn)
                lax.fori_loop(0, PB,
                              functools.partial(_stats_quad, gbufs[b]), 0)
                for tq in range(PB // TQ):
                    lax.fori_loop(tq * TQ, (tq + 1) * TQ,
                                  functools.partial(_norm_quad, gbufs[b]), 0)
                    for s_l in range(SEQ_C):
                        pltpu.async_copy(
                            gbufs[b].at[pl.ds(s_l * PB + tq * TQ, TQ), :],
                            out_hbm.at[c * SEQ_C + s_l,
                                       pl.ds(w * PB + tq * TQ, TQ), :],
                            osem[b])
        return 0

    n_outer = (N_CHUNK + NBUF - 1) // NBUF
    lax.fori_loop(0, n_outer, _outer, 0)

    for b in range(NBUF):
        _out_wait(b)


def kernel(input_ids, word_emb, pos_emb, type_emb, gamma, beta):
    ids_r = (
        input_ids.astype(jnp.int32)
        .reshape(B, NW, PB)
        .transpose(1, 0, 2)
        .reshape(NW * 4, B * PB // 4)
    )
    return _bert_emb(ids_r, word_emb, pos_emb, type_emb, gamma, beta)


# speedup vs baseline: 4.0189x; 1.0064x over previous
import functools

import jax
import jax.numpy as jnp
from jax import lax
from jax.experimental import pallas as pl
from jax.experimental.pallas import tpu as pltpu
from jax.experimental.pallas import tpu_sc as plsc

B = 256
S = 512
H = 768
L = 16
NC, NS = 2, 16
NW = NC * NS
PB = S // NW
SEQ_C = 4
T_C = SEQ_C * PB
N_CHUNK = B // SEQ_C
NBUF = 2
HJ = H // L
TQ = 4
EPS = 1e-12

_mesh = plsc.VectorSubcoreMesh(core_axis_name="c", subcore_axis_name="s")


def _lane_sum2(a, b):
    ii = lax.iota(jnp.int32, L)
    dnums = lax.GatherDimensionNumbers(
        offset_dims=(), collapsed_slice_dims=(0,), start_index_map=(0,))

    def perm(x, idx):
        return lax.gather(x, idx[:, None], dnums, slice_sizes=(1,),
                          mode=lax.GatherScatterMode.PROMISE_IN_BOUNDS)

    for sh in (8, 4, 2, 1):
        idx = lax.bitwise_xor(ii, jnp.int32(sh))
        a = a + perm(a, idx)
        b = b + perm(b, idx)
    return a, b


def _rsqrt(v):
    i = lax.bitcast_convert_type(v, jnp.int32)
    i = jnp.int32(0x5F3759DF) - lax.shift_right_arithmetic(i, jnp.int32(1))
    y = lax.bitcast_convert_type(i, jnp.float32)
    for _ in range(2):
        y = y * (1.5 - 0.5 * v * y * y)
    return y


@functools.partial(
    pl.kernel,
    out_type=jax.ShapeDtypeStruct((B, S, H), jnp.float32),
    mesh=_mesh,
    scratch_types=[
        pltpu.VMEM((B * PB // 4,), jnp.int32),
        [pltpu.VMEM((T_C, H), jnp.float32) for _ in range(NBUF)],
        pltpu.VMEM((2 * T_C, L), jnp.float32),
        pltpu.VMEM((PB, H), jnp.float32),
        pltpu.VMEM((H,), jnp.float32),
        pltpu.VMEM((H,), jnp.float32),
        pltpu.VMEM((H,), jnp.float32),
        [pltpu.SemaphoreType.DMA for _ in range(NBUF)],
        [pltpu.SemaphoreType.DMA for _ in range(NBUF)],
    ],
)
def _bert_emb(ids_hbm, word_hbm, pos_hbm, type_hbm, gamma_hbm, beta_hbm,
              out_hbm, ids_v, gbufs, stats_v, pos_v, type_v,
              gamma_v, beta_v, gsem, osem):
    w = lax.axis_index("s") * NC + lax.axis_index("c")

    pltpu.sync_copy(pos_hbm.at[pl.ds(w * PB, PB), :], pos_v)
    pltpu.sync_copy(type_hbm.at[0], type_v)
    pltpu.sync_copy(gamma_hbm, gamma_v)
    pltpu.sync_copy(beta_hbm, beta_v)
    pltpu.sync_copy(ids_hbm.at[w * 4], ids_v)

    def _add_type(i, _):
        for j in range(HJ):
            sl = pl.ds(j * L, L)
            pos_v[i, sl] = pos_v[i, sl] + type_v[sl]
        return 0

    lax.fori_loop(0, PB, _add_type, 0)

    def _gather_start(c, b):
        off = lax.rem(c, N_CHUNK // 4) * T_C
        idx = ids_v.at[pl.ds(off, T_C)]
        pltpu.async_copy(word_hbm.at[idx], gbufs[b], gsem[b])

    def _gather_wait(b):
        pltpu.make_async_copy(word_hbm.at[ids_v.at[pl.ds(0, T_C)]],
                              gbufs[b], gsem[b]).wait()

    def _out_wait(b):
        for _ in range(SEQ_C * (PB // TQ)):
            pltpu.make_async_copy(
                gbufs[b].at[pl.ds(0, TQ), :],
                out_hbm.at[0, pl.ds(w * PB, TQ), :],
                osem[b]).wait()

    def _stats_quad(rows_v, t, _):
        rows = [t, t + PB, t + 2 * PB, t + 3 * PB]
        ss = [jnp.zeros((L,), jnp.float32) for _ in range(SEQ_C)]
        qs = [jnp.zeros((L,), jnp.float32) for _ in range(SEQ_C)]
        for j in range(HJ):
            sl = pl.ds(j * L, L)
            p = pos_v[t, sl]
            for r in range(SEQ_C):
                x = rows_v[rows[r], sl] + p
                ss[r] = ss[r] + x
                qs[r] = qs[r] + x * x
        ss[0], ss[1] = _lane_sum2(ss[0], ss[1])
        ss[2], ss[3] = _lane_sum2(ss[2], ss[3])
        qs[0], qs[1] = _lane_sum2(qs[0], qs[1])
        qs[2], qs[3] = _lane_sum2(qs[2], qs[3])
        for r in range(SEQ_C):
            m = ss[r] * (1.0 / H)
            a = _rsqrt(qs[r] * (1.0 / H) - m * m + EPS)
            stats_v[2 * rows[r], :] = m
            stats_v[2 * rows[r] + 1, :] = a
        return 0

    QB = 6

    def _norm_quad(rows_v, t, _):
        rows = [t, t + PB, t + 2 * PB, t + 3 * PB]
        ms = [stats_v[2 * r, :] for r in rows]
        as_ = [stats_v[2 * r + 1, :] for r in rows]
        for j0 in range(0, HJ, QB):
            ys = []
            for j in range(j0, j0 + QB):
                sl = pl.ds(j * L, L)
                p = pos_v[t, sl]
                g = gamma_v[sl]
                bt = beta_v[sl]
                for r in range(SEQ_C):
                    ys.append((rows_v[rows[r], sl] + (p - ms[r]))
                              * as_[r] * g + bt)
            for k, j in enumerate(range(j0, j0 + QB)):
                sl = pl.ds(j * L, L)
                for r in range(SEQ_C):
                    rows_v[rows[r], sl] = ys[SEQ_C * k + r]
        return 0

    _gather_start(0, 0)

    def _outer(it, _):
        c0 = it * NBUF
        for b in range(NBUF):
            c = c0 + b
            bn = (b + 1) % NBUF

            @pl.when(c + 1 < N_CHUNK)
            def _():
                @pl.when(c >= 1)
                def _():
                    _out_wait(bn)
                @pl.when(lax.rem(c + 1, N_CHUNK // 4) != 0)
                def _():
                    _gather_start(c + 1, bn)

            @pl.when(c < N_CHUNK)
            def _():
                _gather_wait(b)
                @pl.when(jnp.logical_and(lax.rem(c + 1, N_CHUNK // 4) == 0,
                                         c + 1 < N_CHUNK))
                def _():
                    q = lax.div(c + 1, N_CHUNK // 4)
                    pltpu.sync_copy(ids_hbm.at[w * 4 + q], ids_v)
                    _gather_start(c + 1, bn)
                lax.fori_loop(0, PB,
                              functools.partial(_stats_quad, gbufs[b]), 0)
                for tq in range(PB // TQ):
                    lax.fori_loop(tq * TQ, (tq + 1) * TQ,
                                  functools.partial(_norm_quad, gbufs[b]), 0)
                    for s_l in range(SEQ_C):
                        pltpu.async_copy(
                            gbufs[b].at[pl.ds(s_l * PB + tq * TQ, TQ), :],
                            out_hbm.at[c * SEQ_C + s_l,
                                       pl.ds(w * PB + tq * TQ, TQ), :],
                            osem[b])
        return 0

    n_outer = (N_CHUNK + NBUF - 1) // NBUF
    lax.fori_loop(0, n_outer, _outer, 0)

    for b in range(NBUF):
        _out_wait(b)


def kernel(input_ids, word_emb, pos_emb, type_emb, gamma, beta):
    ids_r = (
        input_ids.astype(jnp.int32)
        .reshape(B, NW, PB)
        .transpose(1, 0, 2)
        .reshape(NW * 4, B * PB // 4)
    )
    return _bert_emb(ids_r, word_emb, pos_emb, type_emb, gamma, beta)


# speedup vs baseline: 4.3247x; 1.0761x over previous
import functools

import jax
import jax.numpy as jnp
from jax import lax
from jax.experimental import pallas as pl
from jax.experimental.pallas import tpu as pltpu
from jax.experimental.pallas import tpu_sc as plsc

B = 256
S = 512
H = 768
L = 16
NC, NS = 2, 16
NW = NC * NS
PB = S // NW
SEQ_C = 4
T_C = SEQ_C * PB
N_CHUNK = B // SEQ_C
NBUF = 2
HJ = H // L
TQ = 8
EPS = 1e-12

_mesh = plsc.VectorSubcoreMesh(core_axis_name="c", subcore_axis_name="s")


def _lane_sum2(a, b):
    ii = lax.iota(jnp.int32, L)
    dnums = lax.GatherDimensionNumbers(
        offset_dims=(), collapsed_slice_dims=(0,), start_index_map=(0,))

    def perm(x, idx):
        return lax.gather(x, idx[:, None], dnums, slice_sizes=(1,),
                          mode=lax.GatherScatterMode.PROMISE_IN_BOUNDS)

    for sh in (8, 4, 2, 1):
        idx = lax.bitwise_xor(ii, jnp.int32(sh))
        a = a + perm(a, idx)
        b = b + perm(b, idx)
    return a, b


def _rsqrt(v):
    i = lax.bitcast_convert_type(v, jnp.int32)
    i = jnp.int32(0x5F3759DF) - lax.shift_right_arithmetic(i, jnp.int32(1))
    y = lax.bitcast_convert_type(i, jnp.float32)
    for _ in range(2):
        y = y * (1.5 - 0.5 * v * y * y)
    return y


@functools.partial(
    pl.kernel,
    out_type=jax.ShapeDtypeStruct((B, S, H), jnp.float32),
    mesh=_mesh,
    scratch_types=[
        pltpu.VMEM((B * PB // 4,), jnp.int32),
        [pltpu.VMEM((T_C, H), jnp.float32) for _ in range(NBUF)],
        pltpu.VMEM((2 * T_C, L), jnp.float32),
        pltpu.VMEM((PB, H), jnp.float32),
        pltpu.VMEM((H,), jnp.float32),
        pltpu.VMEM((H,), jnp.float32),
        pltpu.VMEM((H,), jnp.float32),
        [pltpu.SemaphoreType.DMA for _ in range(NBUF)],
        [pltpu.SemaphoreType.DMA for _ in range(NBUF)],
    ],
)
def _bert_emb(ids_hbm, word_hbm, pos_hbm, type_hbm, gamma_hbm, beta_hbm,
              out_hbm, ids_v, gbufs, stats_v, pos_v, type_v,
              gamma_v, beta_v, gsem, osem):
    w = lax.axis_index("s") * NC + lax.axis_index("c")

    pltpu.sync_copy(pos_hbm.at[pl.ds(w * PB, PB), :], pos_v)
    pltpu.sync_copy(type_hbm.at[0], type_v)
    pltpu.sync_copy(gamma_hbm, gamma_v)
    pltpu.sync_copy(beta_hbm, beta_v)
    pltpu.sync_copy(ids_hbm.at[w * 4], ids_v)

    def _add_type(i, _):
        for j in range(HJ):
            sl = pl.ds(j * L, L)
            pos_v[i, sl] = pos_v[i, sl] + type_v[sl]
        return 0

    lax.fori_loop(0, PB, _add_type, 0)

    def _gather_start(c, b):
        off = lax.rem(c, N_CHUNK // 4) * T_C
        idx = ids_v.at[pl.ds(off, T_C)]
        pltpu.async_copy(word_hbm.at[idx], gbufs[b], gsem[b])

    def _gather_wait(b):
        pltpu.make_async_copy(word_hbm.at[ids_v.at[pl.ds(0, T_C)]],
                              gbufs[b], gsem[b]).wait()

    def _out_wait(b):
        for _ in range(SEQ_C * (PB // TQ)):
            pltpu.make_async_copy(
                gbufs[b].at[pl.ds(0, TQ), :],
                out_hbm.at[0, pl.ds(w * PB, TQ), :],
                osem[b]).wait()

    def _stats_quad(rows_v, t, _):
        rows = [t, t + PB, t + 2 * PB, t + 3 * PB]
        ss = [jnp.zeros((L,), jnp.float32) for _ in range(SEQ_C)]
        qs = [jnp.zeros((L,), jnp.float32) for _ in range(SEQ_C)]
        for j in range(HJ):
            sl = pl.ds(j * L, L)
            p = pos_v[t, sl]
            for r in range(SEQ_C):
                x = rows_v[rows[r], sl] + p
                ss[r] = ss[r] + x
                qs[r] = qs[r] + x * x
        ss[0], ss[1] = _lane_sum2(ss[0], ss[1])
        ss[2], ss[3] = _lane_sum2(ss[2], ss[3])
        qs[0], qs[1] = _lane_sum2(qs[0], qs[1])
        qs[2], qs[3] = _lane_sum2(qs[2], qs[3])
        for r in range(SEQ_C):
            m = ss[r] * (1.0 / H)
            a = _rsqrt(qs[r] * (1.0 / H) - m * m + EPS)
            stats_v[2 * rows[r], :] = m
            stats_v[2 * rows[r] + 1, :] = a
        return 0

    QB = 6

    def _norm_quad(rows_v, t, _):
        rows = [t, t + PB, t + 2 * PB, t + 3 * PB]
        ms = [stats_v[2 * r, :] for r in rows]
        as_ = [stats_v[2 * r + 1, :] for r in rows]
        for j0 in range(0, HJ, QB):
            ys = []
            for j in range(j0, j0 + QB):
                sl = pl.ds(j * L, L)
                p = pos_v[t, sl]
                g = gamma_v[sl]
                bt = beta_v[sl]
                for r in range(SEQ_C):
                    ys.append((rows_v[rows[r], sl] + (p - ms[r]))
                              * as_[r] * g + bt)
            for k, j in enumerate(range(j0, j0 + QB)):
                sl = pl.ds(j * L, L)
                for r in range(SEQ_C):
                    rows_v[rows[r], sl] = ys[SEQ_C * k + r]
        return 0

    _gather_start(0, 0)

    def _outer(it, _):
        c0 = it * NBUF
        for b in range(NBUF):
            c = c0 + b
            bn = (b + 1) % NBUF

            @pl.when(c + 1 < N_CHUNK)
            def _():
                @pl.when(c >= 1)
                def _():
                    _out_wait(bn)
                @pl.when(lax.rem(c + 1, N_CHUNK // 4) != 0)
                def _():
                    _gather_start(c + 1, bn)

            @pl.when(c < N_CHUNK)
            def _():
                _gather_wait(b)
                @pl.when(jnp.logical_and(lax.rem(c + 1, N_CHUNK // 4) == 0,
                                         c + 1 < N_CHUNK))
                def _():
                    q = lax.div(c + 1, N_CHUNK // 4)
                    pltpu.sync_copy(ids_hbm.at[w * 4 + q], ids_v)
                    _gather_start(c + 1, bn)
                lax.fori_loop(0, PB,
                              functools.partial(_stats_quad, gbufs[b]), 0)
                for tq in range(PB // TQ):
                    lax.fori_loop(tq * TQ, (tq + 1) * TQ,
                                  functools.partial(_norm_quad, gbufs[b]), 0)
                    for s_l in range(SEQ_C):
                        pltpu.async_copy(
                            gbufs[b].at[pl.ds(s_l * PB + tq * TQ, TQ), :],
                            out_hbm.at[c * SEQ_C + s_l,
                                       pl.ds(w * PB + tq * TQ, TQ), :],
                            osem[b])
        return 0

    n_outer = (N_CHUNK + NBUF - 1) // NBUF
    lax.fori_loop(0, n_outer, _outer, 0)

    for b in range(NBUF):
        _out_wait(b)


def kernel(input_ids, word_emb, pos_emb, type_emb, gamma, beta):
    ids_r = (
        input_ids.astype(jnp.int32)
        .reshape(B, NW, PB)
        .transpose(1, 0, 2)
        .reshape(NW * 4, B * PB // 4)
    )
    return _bert_emb(ids_r, word_emb, pos_emb, type_emb, gamma, beta)
